# baseline (speedup 1.0000x reference)
import functools

import jax
import jax.numpy as jnp
from jax import lax
from jax.experimental import pallas as pl
from jax.experimental.pallas import tpu as pltpu
from jax.experimental.pallas import tpu_sc as plsc

_B = 16384
_F = 36
_E = 64
_H = 200
_C = 3
_N = _B * _F
_V = 1000000
_QP = 8192
_NBLK = -(-_V // (2 * _QP))
_VT = _NBLK * _QP

_NC, _NS = 2, 16
_NW = _NC * _NS
_NCHUNK = 8
_BCH = _B // _NCHUNK
_SAMP_W = _BCH // _NW
_ROWS_W = _SAMP_W * _F
_SAMP_CH = 8
_ROWS_CH = _SAMP_CH * _F
_STEPS = _SAMP_W // _SAMP_CH


def _prep_kernel(et_ref, o_ref):
    et = et_ref[...]
    o_ref[...] = jnp.concatenate([et[:, :_QP], et[:, _QP:]], axis=0).T


def _tc_prep(embT):
    return pl.pallas_call(
        _prep_kernel,
        grid=(_NBLK,),
        in_specs=[pl.BlockSpec((_E, 2 * _QP), lambda i: (0, i))],
        out_specs=pl.BlockSpec((_QP, 2 * _E), lambda i: (i, 0)),
        out_shape=jax.ShapeDtypeStruct((_VT, 2 * _E), jnp.float32),
    )(embT)


def _sc_gather_pairs(embq, idxp):
    mesh = plsc.VectorSubcoreMesh(core_axis_name="c", subcore_axis_name="s")

    @functools.partial(
        pl.kernel,
        out_type=jax.ShapeDtypeStruct((_BCH, _F * 2 * _E), jnp.float32),
        mesh=mesh,
        scratch_types=[
            pltpu.VMEM((_ROWS_W,), jnp.int32),
            pltpu.VMEM((_ROWS_CH, 2 * _E), jnp.float32),
            pltpu.VMEM((_ROWS_CH, 2 * _E), jnp.float32),
            pltpu.SemaphoreType.DMA,
            pltpu.SemaphoreType.DMA,
            pltpu.SemaphoreType.DMA,
            pltpu.SemaphoreType.DMA,
        ],
    )
    def gather_kernel(table_hbm, idx_hbm, out_hbm, idx_v, rows_a, rows_b,
                      gsem_a, gsem_b, wsem_a, wsem_b):
        wid = lax.axis_index("s") * _NC + lax.axis_index("c")
        row0 = wid * _ROWS_W
        samp0 = wid * _SAMP_W
        pltpu.sync_copy(idx_hbm.at[pl.ds(row0, _ROWS_W)], idx_v)

        @pl.loop(0, _ROWS_W, step=16)
        def _(o):
            v = idx_v[pl.ds(o, 16)]
            idx_v[pl.ds(o, 16)] = ((v >> 14) << 13) | (v & (_QP - 1))

        def g_start(i, rows, gsem):
            idx_c = idx_v.at[pl.ds(i * _ROWS_CH, _ROWS_CH)]
            pltpu.make_async_copy(table_hbm.at[idx_c], rows, gsem).start()

        def g_wait(rows, gsem):
            pltpu.make_async_copy(table_hbm.at[idx_v.at[pl.ds(0, _ROWS_CH)]],
                                  rows, gsem).wait()

        def w_start(i, rows, wsem):
            dst = out_hbm.at[pl.ds(samp0 + i * _SAMP_CH, _SAMP_CH)]
            pltpu.make_async_copy(
                rows.reshape(_SAMP_CH, _F * 2 * _E), dst, wsem).start()

        def w_wait(rows, wsem):
            dst = out_hbm.at[pl.ds(samp0, _SAMP_CH)]
            pltpu.make_async_copy(
                rows.reshape(_SAMP_CH, _F * 2 * _E), dst, wsem).wait()

        g_start(0, rows_a, gsem_a)

        @pl.loop(0, _STEPS // 2)
        def _(k):
            i0 = 2 * k

            @pl.when(k > 0)
            def _():
                w_wait(rows_b, wsem_b)

            g_wait(rows_a, gsem_a)
            g_start(i0 + 1, rows_b, gsem_b)
            w_start(i0, rows_a, wsem_a)
            g_wait(rows_b, gsem_b)
            w_wait(rows_a, wsem_a)

            @pl.when(k < _STEPS // 2 - 1)
            def _():
                g_start(i0 + 2, rows_a, gsem_a)

            w_start(i0 + 1, rows_b, wsem_b)

        w_wait(rows_b, wsem_b)

    return gather_kernel(embq, idxp)


_BM = 512


def _mlp_kernel(x2_ref, t_ref, w1_ref, b1_ref, w2_ref, b2_ref, o_ref):
    x2 = x2_ref[...]
    r = (t_ref[...] >> 13) & 1
    hm = lax.broadcasted_iota(jnp.int32, (1, 2 * _E), 1) >= _E
    parts = []
    for f in range(_F):
        blk = x2[:, 2 * _E * f: 2 * _E * (f + 1)]
        rf = r[:, f: f + 1] != 0
        parts.append(jnp.where(rf == hm, blk, 0.0))
    xs = jnp.concatenate(parts, axis=1).astype(jnp.bfloat16)
    h = jnp.dot(xs, w1_ref[...], preferred_element_type=jnp.float32)
    h = jnp.maximum(h + b1_ref[...], 0.0)
    o_ref[...] = (
        jnp.dot(h, w2_ref[...], preferred_element_type=jnp.float32)
        + b2_ref[...]
    )


def _tc_mlp(x2, r, w1d, b1, w2, b2):
    k2 = _F * 2 * _E
    return pl.pallas_call(
        _mlp_kernel,
        grid=(_BCH // _BM,),
        in_specs=[
            pl.BlockSpec((_BM, k2), lambda i: (i, 0)),
            pl.BlockSpec((_BM, _F), lambda i: (i, 0)),
            pl.BlockSpec((k2, _H), lambda i: (0, 0)),
            pl.BlockSpec((1, _H), lambda i: (0, 0)),
            pl.BlockSpec((_H, _C), lambda i: (0, 0)),
            pl.BlockSpec((1, _C), lambda i: (0, 0)),
        ],
        out_specs=pl.BlockSpec((_BM, _C), lambda i: (i, 0)),
        out_shape=jax.ShapeDtypeStruct((_BCH, _C), jnp.float32),
    )(x2, r, w1d, b1, w2, b2)


def kernel(t, emb, W1, b1, W2, b2):
    embq = _tc_prep(emb.T)
    idx = t.reshape(_N)
    w1d = jnp.repeat(
        W1.reshape(_F, 1, _E, _H), 2, axis=1
    ).reshape(_F * 2 * _E, _H).astype(jnp.bfloat16)
    b1r = b1.reshape(1, _H)
    b2r = b2.reshape(1, _C)
    outs = []
    for c in range(_NCHUNK):
        idx_c = lax.slice(idx, (c * _BCH * _F,), ((c + 1) * _BCH * _F,))
        t_c = lax.slice(t, (c * _BCH, 0), ((c + 1) * _BCH, _F))
        x2c = _sc_gather_pairs(embq, idx_c)
        outs.append(_tc_mlp(x2c, t_c, w1d, b1r, W2, b2r))
    return jnp.concatenate(outs, axis=0)


# speedup vs baseline: 1.0297x; 1.0297x over previous
import functools

import jax
import jax.numpy as jnp
from jax import lax
from jax.experimental import pallas as pl
from jax.experimental.pallas import tpu as pltpu
from jax.experimental.pallas import tpu_sc as plsc

_B = 16384
_F = 36
_E = 64
_H = 200
_C = 3
_N = _B * _F
_V = 1000000
_QP = 16384
_SH = _QP.bit_length() - 1
_NBLK = -(-_V // (2 * _QP))
_VT = _NBLK * _QP

_NC, _NS = 2, 16
_NW = _NC * _NS
_NCHUNK = 4
_BCH = _B // _NCHUNK
_SAMP_W = _BCH // _NW
_ROWS_W = _SAMP_W * _F
_SAMP_CH = 8
_ROWS_CH = _SAMP_CH * _F
_STEPS = _SAMP_W // _SAMP_CH


def _prep_kernel(et_ref, o_ref):
    et = et_ref[...]
    o_ref[...] = jnp.concatenate([et[:, :_QP], et[:, _QP:]], axis=0).T


def _tc_prep(embT):
    return pl.pallas_call(
        _prep_kernel,
        grid=(_NBLK,),
        in_specs=[pl.BlockSpec((_E, 2 * _QP), lambda i: (0, i))],
        out_specs=pl.BlockSpec((_QP, 2 * _E), lambda i: (i, 0)),
        out_shape=jax.ShapeDtypeStruct((_VT, 2 * _E), jnp.float32),
    )(embT)


def _sc_gather_pairs(embq, idxp):
    mesh = plsc.VectorSubcoreMesh(core_axis_name="c", subcore_axis_name="s")

    @functools.partial(
        pl.kernel,
        out_type=jax.ShapeDtypeStruct((_BCH, _F * 2 * _E), jnp.float32),
        mesh=mesh,
        scratch_types=[
            pltpu.VMEM((_ROWS_W,), jnp.int32),
            pltpu.VMEM((_ROWS_CH, 2 * _E), jnp.float32),
            pltpu.VMEM((_ROWS_CH, 2 * _E), jnp.float32),
            pltpu.SemaphoreType.DMA,
            pltpu.SemaphoreType.DMA,
            pltpu.SemaphoreType.DMA,
            pltpu.SemaphoreType.DMA,
        ],
    )
    def gather_kernel(table_hbm, idx_hbm, out_hbm, idx_v, rows_a, rows_b,
                      gsem_a, gsem_b, wsem_a, wsem_b):
        wid = lax.axis_index("s") * _NC + lax.axis_index("c")
        row0 = wid * _ROWS_W
        samp0 = wid * _SAMP_W
        pltpu.sync_copy(idx_hbm.at[pl.ds(row0, _ROWS_W)], idx_v)

        @pl.loop(0, _ROWS_W, step=16)
        def _(o):
            v = idx_v[pl.ds(o, 16)]
            idx_v[pl.ds(o, 16)] = ((v >> (_SH + 1)) << _SH) | (v & (_QP - 1))

        def g_start(i, rows, gsem):
            idx_c = idx_v.at[pl.ds(i * _ROWS_CH, _ROWS_CH)]
            pltpu.make_async_copy(table_hbm.at[idx_c], rows, gsem).start()

        def g_wait(rows, gsem):
            pltpu.make_async_copy(table_hbm.at[idx_v.at[pl.ds(0, _ROWS_CH)]],
                                  rows, gsem).wait()

        def w_start(i, rows, wsem):
            dst = out_hbm.at[pl.ds(samp0 + i * _SAMP_CH, _SAMP_CH)]
            pltpu.make_async_copy(
                rows.reshape(_SAMP_CH, _F * 2 * _E), dst, wsem).start()

        def w_wait(rows, wsem):
            dst = out_hbm.at[pl.ds(samp0, _SAMP_CH)]
            pltpu.make_async_copy(
                rows.reshape(_SAMP_CH, _F * 2 * _E), dst, wsem).wait()

        g_start(0, rows_a, gsem_a)

        @pl.loop(0, _STEPS // 2)
        def _(k):
            i0 = 2 * k

            @pl.when(k > 0)
            def _():
                w_wait(rows_b, wsem_b)

            g_wait(rows_a, gsem_a)
            g_start(i0 + 1, rows_b, gsem_b)
            w_start(i0, rows_a, wsem_a)
            g_wait(rows_b, gsem_b)
            w_wait(rows_a, wsem_a)

            @pl.when(k < _STEPS // 2 - 1)
            def _():
                g_start(i0 + 2, rows_a, gsem_a)

            w_start(i0 + 1, rows_b, wsem_b)

        w_wait(rows_b, wsem_b)

    return gather_kernel(embq, idxp)


_BM = 512


def _mlp_kernel(x2_ref, t_ref, w1_ref, b1_ref, w2_ref, b2_ref, o_ref):
    x2 = x2_ref[...]
    r = (t_ref[...] >> _SH) & 1
    hm = lax.broadcasted_iota(jnp.int32, (1, 2 * _E), 1) >= _E
    parts = []
    for f in range(_F):
        blk = x2[:, 2 * _E * f: 2 * _E * (f + 1)]
        rf = r[:, f: f + 1] != 0
        parts.append(jnp.where(rf == hm, blk, 0.0))
    xs = jnp.concatenate(parts, axis=1).astype(jnp.bfloat16)
    h = jnp.dot(xs, w1_ref[...], preferred_element_type=jnp.float32)
    h = jnp.maximum(h + b1_ref[...], 0.0)
    o_ref[...] = (
        jnp.dot(h, w2_ref[...], preferred_element_type=jnp.float32)
        + b2_ref[...]
    )


def _tc_mlp(x2, r, w1d, b1, w2, b2):
    k2 = _F * 2 * _E
    return pl.pallas_call(
        _mlp_kernel,
        grid=(_BCH // _BM,),
        in_specs=[
            pl.BlockSpec((_BM, k2), lambda i: (i, 0)),
            pl.BlockSpec((_BM, _F), lambda i: (i, 0)),
            pl.BlockSpec((k2, _H), lambda i: (0, 0)),
            pl.BlockSpec((1, _H), lambda i: (0, 0)),
            pl.BlockSpec((_H, _C), lambda i: (0, 0)),
            pl.BlockSpec((1, _C), lambda i: (0, 0)),
        ],
        out_specs=pl.BlockSpec((_BM, _C), lambda i: (i, 0)),
        out_shape=jax.ShapeDtypeStruct((_BCH, _C), jnp.float32),
    )(x2, r, w1d, b1, w2, b2)


def kernel(t, emb, W1, b1, W2, b2):
    embq = _tc_prep(emb.T)
    idx = t.reshape(_N)
    w1d = jnp.repeat(
        W1.reshape(_F, 1, _E, _H), 2, axis=1
    ).reshape(_F * 2 * _E, _H).astype(jnp.bfloat16)
    b1r = b1.reshape(1, _H)
    b2r = b2.reshape(1, _C)
    outs = []
    for c in range(_NCHUNK):
        idx_c = lax.slice(idx, (c * _BCH * _F,), ((c + 1) * _BCH * _F,))
        t_c = lax.slice(t, (c * _BCH, 0), ((c + 1) * _BCH, _F))
        x2c = _sc_gather_pairs(embq, idx_c)
        outs.append(_tc_mlp(x2c, t_c, w1d, b1r, W2, b2r))
    return jnp.concatenate(outs, axis=0)
